# Initial kernel scaffold; baseline (speedup 1.0000x reference)
import jax
import jax.numpy as jnp
from jax.experimental import pallas as pl


def kernel(fts, edge_index, W1, b1, W_lin, b_lin, alpha):
    raise NotImplementedError("write your pallas kernel here")


# speedup vs baseline: 12.8412x; 12.8412x over previous
import functools

import jax
import jax.numpy as jnp
from jax import lax
from jax.experimental import pallas as pl
from jax.experimental.pallas import tpu as pltpu
from jax.experimental.pallas import tpu_sc as plsc

N = 10000
E = 320000
D_IN = 128
D_H = 64
P = 80
NCLASS = 7

NC = 2
NS = 16
NW = NC * NS
EPW = E // NW
CH = 80
NCH = EPW // CH
NPAD = 10240
RPT = NPAD // NS


def _mm_body(x_ref, w_ref, b_ref, o_ref):
    o_ref[...] = (
        jnp.dot(x_ref[...], w_ref[...], preferred_element_type=jnp.float32)
        + b_ref[...]
    )


def _mid_body(y_ref, h_ref, o_ref):
    t = y_ref[0] + y_ref[1] + h_ref[...]
    o_ref[...] = jnp.maximum(t / t[:, D_H:D_H + 1], 0.0)


def _fin_body(z_ref, x_ref, w_ref, b_ref, a_ref, o_ref):
    x = x_ref[...]
    t = z_ref[0] + z_ref[1] + x
    xo = a_ref[0, 0] * x + t / t[:, D_H:D_H + 1]
    o_ref[...] = (
        jnp.dot(xo, w_ref[...], preferred_element_type=jnp.float32)
        + b_ref[...]
    )


def _sc_body(table, gidx, sidx, zeros, out, gv, sv, rows, acc, sem):
    c = lax.axis_index("c")
    s = lax.axis_index("s")
    wid = c * NS + s
    pltpu.sync_copy(zeros.at[pl.ds(s * RPT, RPT)], acc.at[pl.ds(s * RPT, RPT)])
    pltpu.sync_copy(gidx.at[wid], gv)
    pltpu.sync_copy(sidx.at[wid], sv)
    plsc.subcore_barrier()

    def chunk(j, carry):
        pltpu.async_copy(table.at[gv.at[j]], rows, sem).wait()
        pltpu.sync_copy(rows, acc.at[sv.at[j]], add=True)
        return carry

    lax.fori_loop(0, NCH, chunk, 0)
    plsc.subcore_barrier()
    pltpu.sync_copy(acc.at[pl.ds(s * RPT, RPT)],
                    out.at[c, pl.ds(s * RPT, RPT)])


@functools.cache
def _scatter_pass():
    return pl.kernel(
        _sc_body,
        out_type=jax.ShapeDtypeStruct((NC, NPAD, P), jnp.float32),
        mesh=plsc.VectorSubcoreMesh(core_axis_name="c", subcore_axis_name="s",
                                    num_cores=NC, num_subcores=NS),
        scratch_types=[
            pltpu.VMEM((NCH, CH), jnp.int32),
            pltpu.VMEM((NCH, CH), jnp.int32),
            pltpu.VMEM((CH, P), jnp.float32),
            pltpu.VMEM_SHARED((NPAD, P), jnp.float32),
            pltpu.SemaphoreType.DMA,
        ],
        compiler_params=pltpu.CompilerParams(use_tc_tiling_on_sc=False),
    )


_GRID = 10
_R = N // _GRID


def _row_spec(w):
    return pl.BlockSpec((_R, w), lambda i: (i, 0))


def _full_spec(shape):
    return pl.BlockSpec(shape, lambda i: tuple(0 for _ in shape))


_mm_call = pl.pallas_call(
    _mm_body,
    grid=(_GRID,),
    in_specs=[_row_spec(D_IN), _full_spec((D_IN, P)), _full_spec((1, P))],
    out_specs=_row_spec(P),
    out_shape=jax.ShapeDtypeStruct((N, P), jnp.float32),
)

_mid_call = pl.pallas_call(
    _mid_body,
    grid=(_GRID,),
    in_specs=[pl.BlockSpec((NC, _R, P), lambda i: (0, i, 0)), _row_spec(P)],
    out_specs=_row_spec(P),
    out_shape=jax.ShapeDtypeStruct((N, P), jnp.float32),
)

_fin_call = pl.pallas_call(
    _fin_body,
    grid=(_GRID,),
    in_specs=[
        pl.BlockSpec((NC, _R, P), lambda i: (0, i, 0)),
        _row_spec(P),
        _full_spec((P, NCLASS)),
        _full_spec((1, NCLASS)),
        _full_spec((1, 1)),
    ],
    out_specs=_row_spec(NCLASS),
    out_shape=jax.ShapeDtypeStruct((N, NCLASS), jnp.float32),
)


def kernel(fts, edge_index, W1, b1, W_lin, b_lin, alpha):
    src = edge_index[0].reshape(NW, NCH, CH)
    dst = edge_index[1].reshape(NW, NCH, CH)
    W1p = jnp.pad(W1, ((0, 0), (0, P - D_H)))
    b1p = jnp.pad(b1, (0, P - D_H)).at[D_H].set(1.0).reshape(1, P)
    zeros = jnp.zeros((NPAD, P), jnp.float32)

    h_pad = _mm_call(fts, W1p, b1p)
    y = _scatter_pass()(h_pad, src, dst, zeros)
    x_pad = _mid_call(y, h_pad)
    z = _scatter_pass()(x_pad, dst, src, zeros)

    Wp = jnp.pad(W_lin, ((0, P - D_H), (0, 0)))
    out_top = _fin_call(z, x_pad, Wp, b_lin.reshape(1, NCLASS),
                        alpha.reshape(1, 1))
    bottom = jnp.broadcast_to(b_lin.reshape(1, NCLASS), (N, NCLASS))
    return jnp.concatenate([out_top, bottom], axis=0)


# speedup vs baseline: 21.7338x; 1.6925x over previous
import functools

import jax
import jax.numpy as jnp
from jax import lax
from jax.experimental import pallas as pl
from jax.experimental.pallas import tpu as pltpu
from jax.experimental.pallas import tpu_sc as plsc

N = 10000
E = 320000
D_IN = 128
D_H = 64
P = 80
NCLASS = 7

NC = 2
NS = 16
NW = NC * NS
EPW = E // NW
CH = 100
NCH = EPW // CH
NBUF = 5
NGRP = NCH // NBUF
NPAD = 10240
RPT = NPAD // NS


def _mm_body(x_ref, w_ref, b_ref, o_ref):
    o_ref[...] = (
        jnp.dot(x_ref[...], w_ref[...], preferred_element_type=jnp.float32)
        + b_ref[...]
    )


def _mid_body(y_ref, h_ref, o_ref):
    t = y_ref[0] + y_ref[1] + h_ref[...]
    o_ref[...] = jnp.maximum(t / t[:, D_H:D_H + 1], 0.0)


def _fin_body(z_ref, x_ref, w_ref, b_ref, a_ref, o_ref):
    x = x_ref[...]
    t = z_ref[0] + z_ref[1] + x
    xo = a_ref[0, 0] * x + t / t[:, D_H:D_H + 1]
    o_ref[...] = (
        jnp.dot(xo, w_ref[...], preferred_element_type=jnp.float32)
        + b_ref[...]
    )


def _sc_body(table, gidx, sidx, zeros, out, gv, sv, rows, acc, *sems):
    gsem = sems[:NBUF]
    ssem = sems[NBUF:]
    c = lax.axis_index("c")
    s = lax.axis_index("s")
    wid = c * NS + s
    pltpu.sync_copy(zeros.at[pl.ds(s * RPT, RPT)], acc.at[pl.ds(s * RPT, RPT)])
    pltpu.sync_copy(gidx.at[wid], gv)
    pltpu.sync_copy(sidx.at[wid], sv)
    plsc.subcore_barrier()

    def gather_start(j, b):
        pltpu.async_copy(table.at[gv.at[j]], rows.at[b], gsem[b])

    def gather_wait(b):
        pltpu.make_async_copy(table.at[gv.at[0]], rows.at[b], gsem[b]).wait()

    def scatter_start(j, b):
        pltpu.async_copy(rows.at[b], acc.at[sv.at[j]], ssem[b], add=True)

    def scatter_wait(b):
        pltpu.make_async_copy(rows.at[b], acc.at[sv.at[0]], ssem[b]).wait()

    for b in range(NBUF):
        gather_start(b, b)

    def group(g, carry):
        for b in range(NBUF):
            gather_wait(b)
            scatter_start(g * NBUF + b, b)
        for b in range(NBUF):
            scatter_wait(b)
            gather_start((g + 1) * NBUF + b, b)
        return carry

    lax.fori_loop(0, NGRP - 1, group, 0)

    for b in range(NBUF):
        gather_wait(b)
        scatter_start((NGRP - 1) * NBUF + b, b)
    for b in range(NBUF):
        scatter_wait(b)

    plsc.subcore_barrier()
    pltpu.sync_copy(acc.at[pl.ds(s * RPT, RPT)],
                    out.at[c, pl.ds(s * RPT, RPT)])


@functools.cache
def _scatter_pass():
    return pl.kernel(
        _sc_body,
        out_type=jax.ShapeDtypeStruct((NC, NPAD, P), jnp.float32),
        mesh=plsc.VectorSubcoreMesh(core_axis_name="c", subcore_axis_name="s",
                                    num_cores=NC, num_subcores=NS),
        scratch_types=[
            pltpu.VMEM((NCH, CH), jnp.int32),
            pltpu.VMEM((NCH, CH), jnp.int32),
            pltpu.VMEM((NBUF, CH, P), jnp.float32),
            pltpu.VMEM_SHARED((NPAD, P), jnp.float32),
        ] + [pltpu.SemaphoreType.DMA] * (2 * NBUF),
        compiler_params=pltpu.CompilerParams(use_tc_tiling_on_sc=False),
    )


_GRID = 10
_R = N // _GRID


def _row_spec(w):
    return pl.BlockSpec((_R, w), lambda i: (i, 0))


def _full_spec(shape):
    return pl.BlockSpec(shape, lambda i: tuple(0 for _ in shape))


_mm_call = pl.pallas_call(
    _mm_body,
    grid=(_GRID,),
    in_specs=[_row_spec(D_IN), _full_spec((D_IN, P)), _full_spec((1, P))],
    out_specs=_row_spec(P),
    out_shape=jax.ShapeDtypeStruct((N, P), jnp.float32),
)

_mid_call = pl.pallas_call(
    _mid_body,
    grid=(_GRID,),
    in_specs=[pl.BlockSpec((NC, _R, P), lambda i: (0, i, 0)), _row_spec(P)],
    out_specs=_row_spec(P),
    out_shape=jax.ShapeDtypeStruct((N, P), jnp.float32),
)

_fin_call = pl.pallas_call(
    _fin_body,
    grid=(_GRID,),
    in_specs=[
        pl.BlockSpec((NC, _R, P), lambda i: (0, i, 0)),
        _row_spec(P),
        _full_spec((P, NCLASS)),
        _full_spec((1, NCLASS)),
        _full_spec((1, 1)),
    ],
    out_specs=_row_spec(NCLASS),
    out_shape=jax.ShapeDtypeStruct((N, NCLASS), jnp.float32),
)


def kernel(fts, edge_index, W1, b1, W_lin, b_lin, alpha):
    src = edge_index[0].reshape(NW, NCH, CH)
    dst = edge_index[1].reshape(NW, NCH, CH)
    W1p = jnp.pad(W1, ((0, 0), (0, P - D_H)))
    b1p = jnp.pad(b1, (0, P - D_H)).at[D_H].set(1.0).reshape(1, P)
    zeros = jnp.zeros((NPAD, P), jnp.float32)

    h_pad = _mm_call(fts, W1p, b1p)
    y = _scatter_pass()(h_pad, src, dst, zeros)
    x_pad = _mid_call(y, h_pad)
    z = _scatter_pass()(x_pad, dst, src, zeros)

    Wp = jnp.pad(W_lin, ((0, P - D_H), (0, 0)))
    out_top = _fin_call(z, x_pad, Wp, b_lin.reshape(1, NCLASS),
                        alpha.reshape(1, 1))
    bottom = jnp.broadcast_to(b_lin.reshape(1, NCLASS), (N, NCLASS))
    return jnp.concatenate([out_top, bottom], axis=0)
